# baseline (speedup 1.0000x reference)
import functools

import jax
import jax.numpy as jnp
from jax import lax
from jax.experimental import pallas as pl
from jax.experimental.pallas import tpu as pltpu
from jax.experimental.pallas import tpu_sc as plsc

VOCAB = 1000000
D = 64
DPAD = 128
BATCH = 4096
HIST = 200

NC, NS = 2, 16
NW = NC * NS
B_TOT = BATCH * HIST
BPW = B_TOT // NW
CHUNK = 128
NCHUNK = BPW // CHUNK
NBUF = 2

BB = 128


def _gather_body(idx_hbm, table_hbm, out_hbm, idx_v, rows0, rows1, g0, g1):
  rows = (rows0, rows1)
  gsem = (g0, g1)
  c = lax.axis_index("c")
  s = lax.axis_index("s")
  wid = s * NC + c
  pltpu.sync_copy(idx_hbm.at[wid], idx_v)
  base = wid * BPW

  def start_gather(j, b):
    pltpu.make_async_copy(table_hbm.at[idx_v.at[j]], rows[b], gsem[b]).start()

  def wait_gather(b):
    pltpu.make_async_copy(table_hbm.at[idx_v.at[0]], rows[b], gsem[b]).wait()

  for b in range(NBUF):
    start_gather(b, b)

  @pl.loop(0, NCHUNK - NBUF, step=NBUF)
  def _(jj):
    for b in range(NBUF):
      j = jj + b
      wait_gather(b)
      pltpu.sync_copy(rows[b].at[:, pl.ds(0, D)],
                      out_hbm.at[pl.ds(base + j * CHUNK, CHUNK)])
      start_gather(j + NBUF, b)

  for b in range(NBUF):
    j = NCHUNK - NBUF + b
    wait_gather(b)
    pltpu.sync_copy(rows[b].at[:, pl.ds(0, D)],
                      out_hbm.at[pl.ds(base + j * CHUNK, CHUNK)])


TW = 512
VMAIN = (VOCAB // TW) * TW
VTAIL = VOCAB - VMAIN
VOCAB_PAD = VMAIN + DPAD


def _table_body(in_ref, out_ref):
  eye = jnp.eye(D, dtype=jnp.float32)
  xt = lax.dot_general(
      in_ref[...], eye, (((0,), (0,)), ((), ())),
      preferred_element_type=jnp.float32,
      precision=lax.Precision.HIGHEST)
  out_ref[...] = jnp.concatenate(
      [xt, jnp.zeros((TW, DPAD - D), jnp.float32)], axis=1)


def _tail_body(_, tail_ref, out_ref):
  t = tail_ref[...]
  top = jnp.concatenate(
      [t, jnp.zeros((VTAIL, DPAD - D), jnp.float32)], axis=1)
  out_ref[...] = jnp.concatenate(
      [top, jnp.zeros((DPAD - VTAIL, DPAD), jnp.float32)], axis=0)


def _pane_body(in_ref, out_ref):
  x = in_ref[...].reshape(BB, HIST // 2, 2 * D)
  for hp in range(HIST // 2):
    xt = jnp.transpose(x[:, hp, :])
    out_ref[pl.ds(2 * hp, 2)] = xt.reshape(2, D, BB)


@jax.jit
def _embed(x_flat, table):
  tp1 = pl.pallas_call(
      _table_body,
      out_shape=jax.ShapeDtypeStruct((VOCAB_PAD, DPAD), jnp.float32),
      grid=(VMAIN // TW,),
      in_specs=[pl.BlockSpec((D, TW), lambda t: (0, t))],
      out_specs=pl.BlockSpec((TW, DPAD), lambda t: (t, 0)),
  )(table.T)
  table_pad = pl.pallas_call(
      _tail_body,
      out_shape=jax.ShapeDtypeStruct((VOCAB_PAD, DPAD), jnp.float32),
      grid=(1,),
      in_specs=[
          pl.BlockSpec(memory_space=pl.ANY),
          pl.BlockSpec((VTAIL, D), lambda t: (0, 0)),
      ],
      out_specs=pl.BlockSpec((DPAD, DPAD), lambda t: (VMAIN // DPAD, 0)),
      input_output_aliases={0: 0},
  )(tp1, table[VMAIN:])

  mesh = plsc.VectorSubcoreMesh(
      core_axis_name="c", subcore_axis_name="s", num_cores=NC,
      num_subcores=NS)
  run = functools.partial(
      pl.kernel,
      out_type=jax.ShapeDtypeStruct((B_TOT, D), jnp.float32),
      mesh=mesh,
      compiler_params=pltpu.CompilerParams(use_tc_tiling_on_sc=False),
      scratch_types=(
          [pltpu.VMEM((NCHUNK, CHUNK), jnp.int32)]
          + [pltpu.VMEM((CHUNK, DPAD), jnp.float32) for _ in range(NBUF)]
          + [pltpu.SemaphoreType.DMA for _ in range(NBUF)]
      ),
  )(_gather_body)
  flat = run(x_flat, table_pad)

  o2 = flat.reshape(B_TOT // 2, 2 * D)
  o3 = pl.pallas_call(
      _pane_body,
      out_shape=jax.ShapeDtypeStruct((HIST, D, BATCH), jnp.float32),
      grid=(BATCH // BB,),
      in_specs=[pl.BlockSpec((BB * HIST // 2, 2 * D), lambda bb: (bb, 0))],
      out_specs=pl.BlockSpec((HIST, D, BB), lambda bb: (0, 0, bb)),
  )(o2)
  return o3


def kernel(x, table):
  x_flat = x.reshape(NW, NCHUNK, CHUNK).astype(jnp.int32)
  out3 = _embed(x_flat, table)
  return out3.transpose(2, 0, 1)


# speedup vs baseline: 1.9928x; 1.9928x over previous
import functools

import jax
import jax.numpy as jnp
from jax import lax
from jax.experimental import pallas as pl
from jax.experimental.pallas import tpu as pltpu
from jax.experimental.pallas import tpu_sc as plsc

VOCAB = 1000000
D = 64
DPAD = 128
BATCH = 4096
HIST = 200

NC, NS = 2, 16
NW = NC * NS
B_TOT = BATCH * HIST
BPW = B_TOT // NW
CHUNK = 128
NCHUNK = BPW // CHUNK
NBUF = 2

BB = 128


def _gather_body(idx_hbm, table_hbm, out_hbm, idx_v, rows0, rows1, g0, g1):
  rows = (rows0, rows1)
  gsem = (g0, g1)
  c = lax.axis_index("c")
  s = lax.axis_index("s")
  wid = s * NC + c
  pltpu.sync_copy(idx_hbm.at[wid], idx_v)
  base = wid * BPW

  def start_gather(j, b):
    pltpu.make_async_copy(table_hbm.at[idx_v.at[j]], rows[b], gsem[b]).start()

  def wait_gather(b):
    pltpu.make_async_copy(table_hbm.at[idx_v.at[0]], rows[b], gsem[b]).wait()

  for b in range(NBUF):
    start_gather(b, b)

  @pl.loop(0, NCHUNK - NBUF, step=NBUF)
  def _(jj):
    for b in range(NBUF):
      j = jj + b
      wait_gather(b)
      pltpu.sync_copy(rows[b], out_hbm.at[pl.ds(base + j * CHUNK, CHUNK)])
      start_gather(j + NBUF, b)

  for b in range(NBUF):
    j = NCHUNK - NBUF + b
    wait_gather(b)
    pltpu.sync_copy(rows[b].at[:, pl.ds(0, D)],
                      out_hbm.at[pl.ds(base + j * CHUNK, CHUNK)])


TW = 512
VMAIN = (VOCAB // TW) * TW
VTAIL = VOCAB - VMAIN
VOCAB_PAD = VMAIN + DPAD


def _table_body(in_ref, out_ref):
  eye = jnp.eye(D, dtype=jnp.float32)
  xt = lax.dot_general(
      in_ref[...], eye, (((0,), (0,)), ((), ())),
      preferred_element_type=jnp.float32,
      precision=lax.Precision.HIGHEST)
  out_ref[...] = jnp.concatenate(
      [xt, jnp.zeros((TW, DPAD - D), jnp.float32)], axis=1)


def _tail_body(_, tail_ref, out_ref):
  t = tail_ref[...]
  top = jnp.concatenate(
      [t, jnp.zeros((VTAIL, DPAD - D), jnp.float32)], axis=1)
  out_ref[...] = jnp.concatenate(
      [top, jnp.zeros((DPAD - VTAIL, DPAD), jnp.float32)], axis=0)


def _pane_body(in_ref, out_ref):
  x = in_ref[...].reshape(BB, HIST // 2, 2 * D)
  for hp in range(HIST // 2):
    xt = jnp.transpose(x[:, hp, :])
    out_ref[pl.ds(2 * hp, 2)] = xt.reshape(2, D, BB)


@jax.jit
def _embed(x_flat, table):
  mesh = plsc.VectorSubcoreMesh(
      core_axis_name="c", subcore_axis_name="s", num_cores=NC,
      num_subcores=NS)
  run = functools.partial(
      pl.kernel,
      out_type=jax.ShapeDtypeStruct((B_TOT, D), jnp.float32),
      mesh=mesh,
      compiler_params=pltpu.CompilerParams(use_tc_tiling_on_sc=False),
      scratch_types=(
          [pltpu.VMEM((NCHUNK, CHUNK), jnp.int32)]
          + [pltpu.VMEM((CHUNK, D), jnp.float32) for _ in range(NBUF)]
          + [pltpu.SemaphoreType.DMA for _ in range(NBUF)]
      ),
  )(_gather_body)
  flat = run(x_flat, table)

  o2 = flat.reshape(B_TOT // 2, 2 * D)
  o3 = pl.pallas_call(
      _pane_body,
      out_shape=jax.ShapeDtypeStruct((HIST, D, BATCH), jnp.float32),
      grid=(BATCH // BB,),
      in_specs=[pl.BlockSpec((BB * HIST // 2, 2 * D), lambda bb: (bb, 0))],
      out_specs=pl.BlockSpec((HIST, D, BB), lambda bb: (0, 0, bb)),
  )(o2)
  return o3


def kernel(x, table):
  x_flat = x.reshape(NW, NCHUNK, CHUNK).astype(jnp.int32)
  out3 = _embed(x_flat, table)
  return out3.transpose(2, 0, 1)


# speedup vs baseline: 2.0423x; 1.0248x over previous
import functools

import jax
import jax.numpy as jnp
from jax import lax
from jax.experimental import pallas as pl
from jax.experimental.pallas import tpu as pltpu
from jax.experimental.pallas import tpu_sc as plsc

VOCAB = 1000000
D = 64
DPAD = 128
BATCH = 4096
HIST = 200

NC, NS = 2, 16
NW = NC * NS
B_TOT = BATCH * HIST
BPW = B_TOT // NW
CHUNK = 256
NCHUNK = BPW // CHUNK
NBUF = 4

BB = 128


def _gather_body(idx_hbm, table_hbm, out_hbm, idx_v, rows0, rows1, rows2,
                 rows3, g0, g1, g2, g3):
  rows = (rows0, rows1, rows2, rows3)
  gsem = (g0, g1, g2, g3)
  c = lax.axis_index("c")
  s = lax.axis_index("s")
  wid = s * NC + c
  pltpu.sync_copy(idx_hbm.at[wid], idx_v)
  base = wid * BPW

  def start_gather(j, b):
    pltpu.make_async_copy(table_hbm.at[idx_v.at[j]], rows[b], gsem[b]).start()

  def wait_gather(b):
    pltpu.make_async_copy(table_hbm.at[idx_v.at[0]], rows[b], gsem[b]).wait()

  for b in range(NBUF):
    start_gather(b, b)

  @pl.loop(0, NCHUNK - NBUF, step=NBUF)
  def _(jj):
    for b in range(NBUF):
      j = jj + b
      wait_gather(b)
      pltpu.sync_copy(rows[b], out_hbm.at[pl.ds(base + j * CHUNK, CHUNK)])
      start_gather(j + NBUF, b)

  for b in range(NBUF):
    j = NCHUNK - NBUF + b
    wait_gather(b)
    pltpu.sync_copy(rows[b].at[:, pl.ds(0, D)],
                      out_hbm.at[pl.ds(base + j * CHUNK, CHUNK)])


TW = 512
VMAIN = (VOCAB // TW) * TW
VTAIL = VOCAB - VMAIN
VOCAB_PAD = VMAIN + DPAD


def _table_body(in_ref, out_ref):
  eye = jnp.eye(D, dtype=jnp.float32)
  xt = lax.dot_general(
      in_ref[...], eye, (((0,), (0,)), ((), ())),
      preferred_element_type=jnp.float32,
      precision=lax.Precision.HIGHEST)
  out_ref[...] = jnp.concatenate(
      [xt, jnp.zeros((TW, DPAD - D), jnp.float32)], axis=1)


def _tail_body(_, tail_ref, out_ref):
  t = tail_ref[...]
  top = jnp.concatenate(
      [t, jnp.zeros((VTAIL, DPAD - D), jnp.float32)], axis=1)
  out_ref[...] = jnp.concatenate(
      [top, jnp.zeros((DPAD - VTAIL, DPAD), jnp.float32)], axis=0)


def _pane_body(in_ref, out_ref):
  x = in_ref[...].reshape(BB, HIST // 2, 2 * D)
  for hp in range(HIST // 2):
    xt = jnp.transpose(x[:, hp, :])
    out_ref[pl.ds(2 * hp, 2)] = xt.reshape(2, D, BB)


@jax.jit
def _embed(x_flat, table):
  mesh = plsc.VectorSubcoreMesh(
      core_axis_name="c", subcore_axis_name="s", num_cores=NC,
      num_subcores=NS)
  run = functools.partial(
      pl.kernel,
      out_type=jax.ShapeDtypeStruct((B_TOT, D), jnp.float32),
      mesh=mesh,
      compiler_params=pltpu.CompilerParams(use_tc_tiling_on_sc=False),
      scratch_types=(
          [pltpu.VMEM((NCHUNK, CHUNK), jnp.int32)]
          + [pltpu.VMEM((CHUNK, D), jnp.float32) for _ in range(NBUF)]
          + [pltpu.SemaphoreType.DMA for _ in range(NBUF)]
      ),
  )(_gather_body)
  flat = run(x_flat, table)

  o2 = flat.reshape(B_TOT // 2, 2 * D)
  o3 = pl.pallas_call(
      _pane_body,
      out_shape=jax.ShapeDtypeStruct((HIST, D, BATCH), jnp.float32),
      grid=(BATCH // BB,),
      in_specs=[pl.BlockSpec((BB * HIST // 2, 2 * D), lambda bb: (bb, 0))],
      out_specs=pl.BlockSpec((HIST, D, BB), lambda bb: (0, 0, bb)),
  )(o2)
  return o3


def kernel(x, table):
  x_flat = x.reshape(NW, NCHUNK, CHUNK).astype(jnp.int32)
  out3 = _embed(x_flat, table)
  return out3.transpose(2, 0, 1)


# speedup vs baseline: 2.0453x; 1.0015x over previous
import functools

import jax
import jax.numpy as jnp
from jax import lax
from jax.experimental import pallas as pl
from jax.experimental.pallas import tpu as pltpu
from jax.experimental.pallas import tpu_sc as plsc

VOCAB = 1000000
D = 64
DPAD = 128
BATCH = 4096
HIST = 200

NC, NS = 2, 16
NW = NC * NS
B_TOT = BATCH * HIST
BPW = B_TOT // NW
CHUNK = 512
NCHUNK = BPW // CHUNK
NBUF = 2

BB = 128


def _gather_body(idx_hbm, table_hbm, out_hbm, idx_v, rows0, rows1, g0, g1):
  rows = (rows0, rows1)
  gsem = (g0, g1)
  c = lax.axis_index("c")
  s = lax.axis_index("s")
  wid = s * NC + c
  pltpu.sync_copy(idx_hbm.at[wid], idx_v)
  base = wid * BPW

  def start_gather(j, b):
    pltpu.make_async_copy(table_hbm.at[idx_v.at[j]], rows[b], gsem[b]).start()

  def wait_gather(b):
    pltpu.make_async_copy(table_hbm.at[idx_v.at[0]], rows[b], gsem[b]).wait()

  for b in range(NBUF):
    start_gather(b, b)

  @pl.loop(0, NCHUNK - NBUF, step=NBUF)
  def _(jj):
    for b in range(NBUF):
      j = jj + b
      wait_gather(b)
      pltpu.sync_copy(rows[b], out_hbm.at[pl.ds(base + j * CHUNK, CHUNK)])
      start_gather(j + NBUF, b)

  for b in range(NBUF):
    j = NCHUNK - NBUF + b
    wait_gather(b)
    pltpu.sync_copy(rows[b].at[:, pl.ds(0, D)],
                      out_hbm.at[pl.ds(base + j * CHUNK, CHUNK)])


TW = 512
VMAIN = (VOCAB // TW) * TW
VTAIL = VOCAB - VMAIN
VOCAB_PAD = VMAIN + DPAD


def _table_body(in_ref, out_ref):
  eye = jnp.eye(D, dtype=jnp.float32)
  xt = lax.dot_general(
      in_ref[...], eye, (((0,), (0,)), ((), ())),
      preferred_element_type=jnp.float32,
      precision=lax.Precision.HIGHEST)
  out_ref[...] = jnp.concatenate(
      [xt, jnp.zeros((TW, DPAD - D), jnp.float32)], axis=1)


def _tail_body(_, tail_ref, out_ref):
  t = tail_ref[...]
  top = jnp.concatenate(
      [t, jnp.zeros((VTAIL, DPAD - D), jnp.float32)], axis=1)
  out_ref[...] = jnp.concatenate(
      [top, jnp.zeros((DPAD - VTAIL, DPAD), jnp.float32)], axis=0)


def _pane_body(in_ref, out_ref):
  x = in_ref[...].reshape(BB, HIST // 2, 2 * D)
  for hp in range(HIST // 2):
    xt = jnp.transpose(x[:, hp, :])
    out_ref[pl.ds(2 * hp, 2)] = xt.reshape(2, D, BB)


@jax.jit
def _embed(x_flat, table):
  mesh = plsc.VectorSubcoreMesh(
      core_axis_name="c", subcore_axis_name="s", num_cores=NC,
      num_subcores=NS)
  run = functools.partial(
      pl.kernel,
      out_type=jax.ShapeDtypeStruct((B_TOT, D), jnp.float32),
      mesh=mesh,
      compiler_params=pltpu.CompilerParams(use_tc_tiling_on_sc=False),
      scratch_types=(
          [pltpu.VMEM((NCHUNK, CHUNK), jnp.int32)]
          + [pltpu.VMEM((CHUNK, D), jnp.float32) for _ in range(NBUF)]
          + [pltpu.SemaphoreType.DMA for _ in range(NBUF)]
      ),
  )(_gather_body)
  flat = run(x_flat, table)

  o2 = flat.reshape(B_TOT // 2, 2 * D)
  o3 = pl.pallas_call(
      _pane_body,
      out_shape=jax.ShapeDtypeStruct((HIST, D, BATCH), jnp.float32),
      grid=(BATCH // BB,),
      in_specs=[pl.BlockSpec((BB * HIST // 2, 2 * D), lambda bb: (bb, 0))],
      out_specs=pl.BlockSpec((HIST, D, BB), lambda bb: (0, 0, bb)),
  )(o2)
  return o3


def kernel(x, table):
  x_flat = x.reshape(NW, NCHUNK, CHUNK).astype(jnp.int32)
  out3 = _embed(x_flat, table)
  return out3.transpose(2, 0, 1)
